# baseline (speedup 1.0000x reference)
import functools

import jax
import jax.numpy as jnp
from jax import lax
from jax.experimental import pallas as pl
from jax.experimental.pallas import tpu as pltpu
from jax.experimental.pallas import tpu_sc as plsc

_N = 10000
_E = 320000
_P = 10240
_NT = 16
_NC = 2
_K = 100
_W = 128

_NB = _E // _NT // _K
_NB2 = _E // (_NC * _NT) // _K
_GB = 50
_NG16 = _NB // _GB
_NG32 = _NB2 // _GB
_RPT = _P // _NT


@functools.cache
def _get_mesh():
    return plsc.VectorSubcoreMesh(core_axis_name="c", subcore_axis_name="s",
                                  num_cores=_NC, num_subcores=_NT)


def _sweep(hs_c, acc, srcb, dstb, rows0, rows1, sem0, sem1, nb):
    pltpu.async_copy(hs_c.at[srcb.at[0]], rows0, sem0)
    half = nb // 2

    def pair(t, carry):
        i0 = 2 * t
        pltpu.async_copy(hs_c.at[srcb.at[i0 + 1]], rows1, sem1)
        pltpu.make_async_copy(hs_c.at[srcb.at[i0]], rows0, sem0).wait()
        pltpu.sync_copy(rows0, acc.at[dstb.at[i0]], add=True)

        @pl.when(t + 1 < half)
        def _():
            pltpu.async_copy(hs_c.at[srcb.at[i0 + 2]], rows0, sem0)

        pltpu.make_async_copy(hs_c.at[srcb.at[i0 + 1]], rows1, sem1).wait()
        pltpu.sync_copy(rows1, acc.at[dstb.at[i0 + 1]], add=True)
        return carry

    lax.fori_loop(0, half, pair, 0)


@functools.cache
def _make_degree():
    def body(dst_hbm, ones_hbm, zeros_hbm, out_hbm, dstb, onesb, acc):
        cid = lax.axis_index("c")
        sid = lax.axis_index("s")
        wid = cid * _NT + sid
        pltpu.sync_copy(ones_hbm, onesb)
        r0 = sid * _RPT
        pltpu.sync_copy(zeros_hbm.at[pl.ds(r0, _RPT), :], acc.at[pl.ds(r0, _RPT), :])
        plsc.subcore_barrier()

        def step(i, carry):
            pltpu.sync_copy(onesb, acc.at[dstb.at[i]], add=True)
            return carry

        for g in range(_NG32):
            pltpu.sync_copy(dst_hbm.at[wid, g], dstb)
            lax.fori_loop(0, _GB, step, 0)
        plsc.subcore_barrier()
        pltpu.sync_copy(acc.at[pl.ds(r0, _RPT), :], out_hbm.at[cid, pl.ds(r0, _RPT), :])

    return pl.kernel(
        body,
        out_type=jax.ShapeDtypeStruct((_NC, _P, _W), jnp.float32),
        mesh=_get_mesh(),
        scratch_types=[
            pltpu.VMEM((_GB, _K), jnp.int32),
            pltpu.VMEM((_K, _W), jnp.float32),
            pltpu.VMEM_SHARED((_P, _W), jnp.float32),
        ],
    )


@functools.cache
def _make_agg_split():
    def body(hs_hbm, zeros_hbm, src_hbm, dst_hbm, out_hbm,
             srcb, dstb, rows0, rows1, acc, sem0, sem1):
        cid = lax.axis_index("c")
        sid = lax.axis_index("s")
        wid = cid * _NT + sid
        r0 = sid * _RPT

        @pl.when(cid == 0)
        def _():
            pltpu.sync_copy(hs_hbm.at[pl.ds(r0, _RPT), :], acc.at[pl.ds(r0, _RPT), :])

        @pl.when(cid == 1)
        def _():
            pltpu.sync_copy(zeros_hbm.at[pl.ds(r0, _RPT), :], acc.at[pl.ds(r0, _RPT), :])

        plsc.subcore_barrier()
        for g in range(_NG32):
            pltpu.sync_copy(src_hbm.at[wid, g], srcb)
            pltpu.sync_copy(dst_hbm.at[wid, g], dstb)
            _sweep(hs_hbm, acc, srcb, dstb, rows0, rows1, sem0, sem1, _GB)
        plsc.subcore_barrier()
        pltpu.sync_copy(acc.at[pl.ds(r0, _RPT), :], out_hbm.at[cid, pl.ds(r0, _RPT), :])

    return pl.kernel(
        body,
        out_type=jax.ShapeDtypeStruct((_NC, _P, _W), jnp.float32),
        mesh=_get_mesh(),
        scratch_types=[
            pltpu.VMEM((_GB, _K), jnp.int32),
            pltpu.VMEM((_GB, _K), jnp.int32),
            pltpu.VMEM((_K, _W), jnp.float32),
            pltpu.VMEM((_K, _W), jnp.float32),
            pltpu.VMEM_SHARED((_P, _W), jnp.float32),
            pltpu.SemaphoreType.DMA,
            pltpu.SemaphoreType.DMA,
        ],
    )


@functools.cache
def _make_agg(C):
    Cpc = C // _NC

    def body(*refs):
        hs = refs[0:C]
        src_hbm = refs[C]
        dst_hbm = refs[C + 1]
        outs = refs[C + 2: 2 * C + 2]
        srcb, dstb, rows0, rows1, acc, sem0, sem1 = refs[2 * C + 2:]
        cid = lax.axis_index("c")
        sid = lax.axis_index("s")
        r0 = sid * _RPT

        def do_chunk(hs_c, out_c):
            pltpu.sync_copy(hs_c.at[pl.ds(r0, _RPT), :], acc.at[pl.ds(r0, _RPT), :])
            plsc.subcore_barrier()
            for g in range(_NG16):
                pltpu.sync_copy(src_hbm.at[sid, g], srcb)
                pltpu.sync_copy(dst_hbm.at[sid, g], dstb)
                _sweep(hs_c, acc, srcb, dstb, rows0, rows1, sem0, sem1, _GB)
            plsc.subcore_barrier()
            pltpu.sync_copy(acc.at[pl.ds(r0, _RPT), :], out_c.at[pl.ds(r0, _RPT), :])

        for core in range(_NC):
            @pl.when(cid == core)
            def _(core=core):
                for j in range(Cpc):
                    do_chunk(hs[core * Cpc + j], outs[core * Cpc + j])

    return pl.kernel(
        body,
        out_type=tuple(jax.ShapeDtypeStruct((_P, _W), jnp.float32) for _ in range(C)),
        mesh=_get_mesh(),
        scratch_types=[
            pltpu.VMEM((_GB, _K), jnp.int32),
            pltpu.VMEM((_GB, _K), jnp.int32),
            pltpu.VMEM((_K, _W), jnp.float32),
            pltpu.VMEM((_K, _W), jnp.float32),
            pltpu.VMEM_SHARED((_P, _W), jnp.float32),
            pltpu.SemaphoreType.DMA,
            pltpu.SemaphoreType.DMA,
        ],
    )


_BN = 512
_G = _P // _BN


def _tc_pre(x, degp):
    def body(x_ref, d_ref, o_hs, o_dinv):
        deg = d_ref[0][:, :1] + d_ref[1][:, :1] + 1.0
        dinv = lax.rsqrt(deg)
        o_dinv[...] = dinv
        o_hs[...] = x_ref[...] * dinv

    return pl.pallas_call(
        body,
        grid=(_G,),
        in_specs=[
            pl.BlockSpec((_BN, 128), lambda i: (i, 0)),
            pl.BlockSpec((_NC, _BN, _W), lambda i: (0, i, 0)),
        ],
        out_specs=[
            pl.BlockSpec((_BN, 128), lambda i: (i, 0)),
            pl.BlockSpec((_BN, 1), lambda i: (i, 0)),
        ],
        out_shape=[
            jax.ShapeDtypeStruct((_P, 128), jnp.float32),
            jax.ShapeDtypeStruct((_P, 1), jnp.float32),
        ],
    )(x, degp)


def _ff(h, w_ref, b_ref):
    h = lax.dot_general(h, w_ref[...], (((1,), (1,)), ((), ())),
                        precision=lax.Precision.HIGHEST,
                        preferred_element_type=jnp.float32)
    return jnp.maximum(h + b_ref[...], 0.0)


def _tc_conv(a_parts, dinv, Wt, bt, C_out, combine):
    C_in = len(a_parts)
    d_out, d_in = Wt.shape
    W_in = d_in if combine == "sum" else d_in // C_in

    def body(*refs):
        a_refs = refs[:C_in]
        d_ref, w_ref, b_ref = refs[C_in:C_in + 3]
        outs = refs[C_in + 3:]
        if combine == "sum":
            a = a_refs[0][0] + a_refs[0][1]
        else:
            a = jnp.concatenate([r[...] for r in a_refs], axis=1)
        dinv = d_ref[...]
        h = _ff(a * dinv, w_ref, b_ref) * dinv
        for c, o in enumerate(outs):
            o[...] = h[:, c * _W:(c + 1) * _W]

    if combine == "sum":
        a_specs = [pl.BlockSpec((_NC, _BN, W_in), lambda i: (0, i, 0))]
    else:
        a_specs = [pl.BlockSpec((_BN, W_in), lambda i: (i, 0))] * C_in
    in_specs = a_specs + [
        pl.BlockSpec((_BN, 1), lambda i: (i, 0)),
        pl.BlockSpec((d_out, d_in), lambda i: (0, 0)),
        pl.BlockSpec((1, d_out), lambda i: (0, 0)),
    ]
    return pl.pallas_call(
        body,
        grid=(_G,),
        in_specs=in_specs,
        out_specs=[pl.BlockSpec((_BN, _W), lambda i: (i, 0))] * C_out,
        out_shape=[jax.ShapeDtypeStruct((_P, _W), jnp.float32)] * C_out,
    )(*a_parts, dinv, Wt, bt.reshape(1, -1))


def _tc_final(a_chunks, dinv, Wc3, bc3, Wl1, bl1, Wl2, bl2, Wl3, bl3):
    def body(a0, a1, a2, a3, d_ref, w3, b3, w1, b1, w2, b2, wl, bl, o):
        a = jnp.concatenate([a0[...], a1[...], a2[...], a3[...]], axis=1)
        h = _ff(a * d_ref[...], w3, b3)
        h = _ff(h, w1, b1)
        h = _ff(h, w2, b2)
        o[...] = _ff(h, wl, bl)

    def wspec(shape):
        return pl.BlockSpec(shape, lambda i: (0, 0))

    in_specs = (
        [pl.BlockSpec((_BN, _W), lambda i: (i, 0))] * 4 + [
            pl.BlockSpec((_BN, 1), lambda i: (i, 0)),
            wspec((1024, 512)), wspec((1, 1024)),
            wspec((512, 1024)), wspec((1, 512)),
            wspec((256, 512)), wspec((1, 256)),
            wspec((128, 256)), wspec((1, 128)),
        ]
    )
    return pl.pallas_call(
        body,
        grid=(_G,),
        in_specs=in_specs,
        out_specs=pl.BlockSpec((_BN, 128), lambda i: (i, 0)),
        out_shape=jax.ShapeDtypeStruct((_P, 128), jnp.float32),
    )(*a_chunks, dinv, Wc3, bc3.reshape(1, -1), Wl1, bl1.reshape(1, -1),
      Wl2, bl2.reshape(1, -1), Wl3, bl3.reshape(1, -1))


def kernel(x, edge_index, Wc1, bc1, Wc2, bc2, Wc3, bc3,
           Wl1, bl1, Wl2, bl2, Wl3, bl3):
    src16 = edge_index[0].reshape(_NT, _NG16, _GB, _K)
    dst16 = edge_index[1].reshape(_NT, _NG16, _GB, _K)
    src32 = edge_index[0].reshape(_NC * _NT, _NG32, _GB, _K)
    dst32 = edge_index[1].reshape(_NC * _NT, _NG32, _GB, _K)
    ones = jnp.ones((_K, _W), jnp.float32)
    zeros = jnp.zeros((_P, _W), jnp.float32)
    xp = jnp.pad(x, ((0, _P - _N), (0, 0)))

    degp = _make_degree()(dst32, ones, zeros)
    hs1, dinv = _tc_pre(xp, degp)

    a1 = _make_agg_split()(hs1, zeros, src32, dst32)
    hs2 = _tc_conv([a1], dinv, Wc1, bc1, 2, "sum")
    a2 = _make_agg(2)(hs2[0], hs2[1], src16, dst16)
    hs3 = _tc_conv(a2, dinv, Wc2, bc2, 4, "cat")
    a3 = _make_agg(4)(hs3[0], hs3[1], hs3[2], hs3[3], src16, dst16)
    out = _tc_final(a3, dinv, Wc3, bc3, Wl1, bl1, Wl2, bl2, Wl3, bl3)
    return out[:_N]


# speedup vs baseline: 1.1658x; 1.1658x over previous
import functools

import jax
import jax.numpy as jnp
from jax import lax
from jax.experimental import pallas as pl
from jax.experimental.pallas import tpu as pltpu
from jax.experimental.pallas import tpu_sc as plsc

_N = 10000
_E = 320000
_P = 10240
_NT = 16
_NC = 2
_K = 100
_W = 128

_NB = _E // _NT // _K
_NB2 = _E // (_NC * _NT) // _K
_GB = 50
_NG16 = _NB // _GB
_NG32 = _NB2 // _GB
_RPT = _P // _NT


@functools.cache
def _get_mesh():
    return plsc.VectorSubcoreMesh(core_axis_name="c", subcore_axis_name="s",
                                  num_cores=_NC, num_subcores=_NT)


def _sweep(hs_c, acc, srcb, dstb, rows0, rows1, sem0, sem1, nb):
    pltpu.async_copy(hs_c.at[srcb.at[0]], rows0, sem0)
    half = nb // 2

    def pair(t, carry):
        i0 = 2 * t
        pltpu.async_copy(hs_c.at[srcb.at[i0 + 1]], rows1, sem1)
        pltpu.make_async_copy(hs_c.at[srcb.at[i0]], rows0, sem0).wait()
        pltpu.sync_copy(rows0, acc.at[dstb.at[i0]], add=True)

        @pl.when(t + 1 < half)
        def _():
            pltpu.async_copy(hs_c.at[srcb.at[i0 + 2]], rows0, sem0)

        pltpu.make_async_copy(hs_c.at[srcb.at[i0 + 1]], rows1, sem1).wait()
        pltpu.sync_copy(rows1, acc.at[dstb.at[i0 + 1]], add=True)
        return carry

    lax.fori_loop(0, half, pair, 0)


@functools.cache
def _make_degree():
    def body(dst_hbm, ones_hbm, zeros_hbm, out_hbm, dstb, onesb, acc):
        cid = lax.axis_index("c")
        sid = lax.axis_index("s")
        wid = cid * _NT + sid
        pltpu.sync_copy(ones_hbm, onesb)
        r0 = sid * _RPT
        pltpu.sync_copy(zeros_hbm.at[pl.ds(r0, _RPT), :], acc.at[pl.ds(r0, _RPT), :])
        plsc.subcore_barrier()

        def step(i, carry):
            pltpu.sync_copy(onesb, acc.at[dstb.at[i]], add=True)
            return carry

        for g in range(_NG32):
            pltpu.sync_copy(dst_hbm.at[wid, g], dstb)
            lax.fori_loop(0, _GB, step, 0)
        plsc.subcore_barrier()
        pltpu.sync_copy(acc.at[pl.ds(r0, _RPT), :], out_hbm.at[cid, pl.ds(r0, _RPT), :])

    return pl.kernel(
        body,
        out_type=jax.ShapeDtypeStruct((_NC, _P, _W), jnp.float32),
        mesh=_get_mesh(),
        scratch_types=[
            pltpu.VMEM((_GB, _K), jnp.int32),
            pltpu.VMEM((_K, _W), jnp.float32),
            pltpu.VMEM_SHARED((_P, _W), jnp.float32),
        ],
    )


@functools.cache
def _make_agg_split():
    def body(hs_hbm, zeros_hbm, src_hbm, dst_hbm, out_hbm,
             srcb, dstb, rows0, rows1, acc, sem0, sem1):
        cid = lax.axis_index("c")
        sid = lax.axis_index("s")
        wid = cid * _NT + sid
        r0 = sid * _RPT

        @pl.when(cid == 0)
        def _():
            pltpu.sync_copy(hs_hbm.at[pl.ds(r0, _RPT), :], acc.at[pl.ds(r0, _RPT), :])

        @pl.when(cid == 1)
        def _():
            pltpu.sync_copy(zeros_hbm.at[pl.ds(r0, _RPT), :], acc.at[pl.ds(r0, _RPT), :])

        plsc.subcore_barrier()
        for g in range(_NG32):
            pltpu.sync_copy(src_hbm.at[wid, g], srcb)
            pltpu.sync_copy(dst_hbm.at[wid, g], dstb)
            _sweep(hs_hbm, acc, srcb, dstb, rows0, rows1, sem0, sem1, _GB)
        plsc.subcore_barrier()
        pltpu.sync_copy(acc.at[pl.ds(r0, _RPT), :], out_hbm.at[cid, pl.ds(r0, _RPT), :])

    return pl.kernel(
        body,
        out_type=jax.ShapeDtypeStruct((_NC, _P, _W), jnp.float32),
        mesh=_get_mesh(),
        scratch_types=[
            pltpu.VMEM((_GB, _K), jnp.int32),
            pltpu.VMEM((_GB, _K), jnp.int32),
            pltpu.VMEM((_K, _W), jnp.float32),
            pltpu.VMEM((_K, _W), jnp.float32),
            pltpu.VMEM_SHARED((_P, _W), jnp.float32),
            pltpu.SemaphoreType.DMA,
            pltpu.SemaphoreType.DMA,
        ],
    )


@functools.cache
def _make_agg(C):
    Cpc = C // _NC

    def body(*refs):
        hs = refs[0:C]
        src_hbm = refs[C]
        dst_hbm = refs[C + 1]
        outs = refs[C + 2: 2 * C + 2]
        srcb, dstb, rows0, rows1, acc, sem0, sem1 = refs[2 * C + 2:]
        cid = lax.axis_index("c")
        sid = lax.axis_index("s")
        r0 = sid * _RPT

        def do_chunk(hs_c, out_c):
            pltpu.sync_copy(hs_c.at[pl.ds(r0, _RPT), :], acc.at[pl.ds(r0, _RPT), :])
            plsc.subcore_barrier()
            for g in range(_NG16):
                pltpu.sync_copy(src_hbm.at[sid, g], srcb)
                pltpu.sync_copy(dst_hbm.at[sid, g], dstb)
                _sweep(hs_c, acc, srcb, dstb, rows0, rows1, sem0, sem1, _GB)
            plsc.subcore_barrier()
            pltpu.sync_copy(acc.at[pl.ds(r0, _RPT), :], out_c.at[pl.ds(r0, _RPT), :])

        for core in range(_NC):
            @pl.when(cid == core)
            def _(core=core):
                for j in range(Cpc):
                    do_chunk(hs[core * Cpc + j], outs[core * Cpc + j])

    return pl.kernel(
        body,
        out_type=tuple(jax.ShapeDtypeStruct((_P, _W), jnp.float32) for _ in range(C)),
        mesh=_get_mesh(),
        scratch_types=[
            pltpu.VMEM((_GB, _K), jnp.int32),
            pltpu.VMEM((_GB, _K), jnp.int32),
            pltpu.VMEM((_K, _W), jnp.float32),
            pltpu.VMEM((_K, _W), jnp.float32),
            pltpu.VMEM_SHARED((_P, _W), jnp.float32),
            pltpu.SemaphoreType.DMA,
            pltpu.SemaphoreType.DMA,
        ],
    )


_BN = 512
_G = _P // _BN


def _tc_pre(x, degp):
    def body(x_ref, d_ref, o_hs, o_dinv):
        deg = d_ref[0][:, :1] + d_ref[1][:, :1] + 1.0
        dinv = lax.rsqrt(deg)
        o_dinv[...] = dinv
        o_hs[...] = x_ref[...] * dinv

    return pl.pallas_call(
        body,
        grid=(_G,),
        in_specs=[
            pl.BlockSpec((_BN, 128), lambda i: (i, 0)),
            pl.BlockSpec((_NC, _BN, _W), lambda i: (0, i, 0)),
        ],
        out_specs=[
            pl.BlockSpec((_BN, 128), lambda i: (i, 0)),
            pl.BlockSpec((_BN, 1), lambda i: (i, 0)),
        ],
        out_shape=[
            jax.ShapeDtypeStruct((_P, 128), jnp.float32),
            jax.ShapeDtypeStruct((_P, 1), jnp.float32),
        ],
    )(x, degp)


def _ff(h, w_ref, b_ref):
    h = lax.dot_general(h, w_ref[...], (((1,), (1,)), ((), ())),
                        preferred_element_type=jnp.float32)
    return jnp.maximum(h + b_ref[...], 0.0)


def _tc_conv(a_parts, dinv, Wt, bt, C_out, combine):
    C_in = len(a_parts)
    d_out, d_in = Wt.shape
    W_in = d_in if combine == "sum" else d_in // C_in

    def body(*refs):
        a_refs = refs[:C_in]
        d_ref, w_ref, b_ref = refs[C_in:C_in + 3]
        outs = refs[C_in + 3:]
        if combine == "sum":
            a = a_refs[0][0] + a_refs[0][1]
        else:
            a = jnp.concatenate([r[...] for r in a_refs], axis=1)
        dinv = d_ref[...]
        h = _ff(a * dinv, w_ref, b_ref) * dinv
        for c, o in enumerate(outs):
            o[...] = h[:, c * _W:(c + 1) * _W]

    if combine == "sum":
        a_specs = [pl.BlockSpec((_NC, _BN, W_in), lambda i: (0, i, 0))]
    else:
        a_specs = [pl.BlockSpec((_BN, W_in), lambda i: (i, 0))] * C_in
    in_specs = a_specs + [
        pl.BlockSpec((_BN, 1), lambda i: (i, 0)),
        pl.BlockSpec((d_out, d_in), lambda i: (0, 0)),
        pl.BlockSpec((1, d_out), lambda i: (0, 0)),
    ]
    return pl.pallas_call(
        body,
        grid=(_G,),
        in_specs=in_specs,
        out_specs=[pl.BlockSpec((_BN, _W), lambda i: (i, 0))] * C_out,
        out_shape=[jax.ShapeDtypeStruct((_P, _W), jnp.float32)] * C_out,
    )(*a_parts, dinv, Wt, bt.reshape(1, -1))


def _tc_final(a_chunks, dinv, Wc3, bc3, Wl1, bl1, Wl2, bl2, Wl3, bl3):
    def body(a0, a1, a2, a3, d_ref, w3, b3, w1, b1, w2, b2, wl, bl, o):
        a = jnp.concatenate([a0[...], a1[...], a2[...], a3[...]], axis=1)
        h = _ff(a * d_ref[...], w3, b3)
        h = _ff(h, w1, b1)
        h = _ff(h, w2, b2)
        o[...] = _ff(h, wl, bl)

    def wspec(shape):
        return pl.BlockSpec(shape, lambda i: (0, 0))

    in_specs = (
        [pl.BlockSpec((_BN, _W), lambda i: (i, 0))] * 4 + [
            pl.BlockSpec((_BN, 1), lambda i: (i, 0)),
            wspec((1024, 512)), wspec((1, 1024)),
            wspec((512, 1024)), wspec((1, 512)),
            wspec((256, 512)), wspec((1, 256)),
            wspec((128, 256)), wspec((1, 128)),
        ]
    )
    return pl.pallas_call(
        body,
        grid=(_G,),
        in_specs=in_specs,
        out_specs=pl.BlockSpec((_BN, 128), lambda i: (i, 0)),
        out_shape=jax.ShapeDtypeStruct((_P, 128), jnp.float32),
    )(*a_chunks, dinv, Wc3, bc3.reshape(1, -1), Wl1, bl1.reshape(1, -1),
      Wl2, bl2.reshape(1, -1), Wl3, bl3.reshape(1, -1))


def kernel(x, edge_index, Wc1, bc1, Wc2, bc2, Wc3, bc3,
           Wl1, bl1, Wl2, bl2, Wl3, bl3):
    src16 = edge_index[0].reshape(_NT, _NG16, _GB, _K)
    dst16 = edge_index[1].reshape(_NT, _NG16, _GB, _K)
    src32 = edge_index[0].reshape(_NC * _NT, _NG32, _GB, _K)
    dst32 = edge_index[1].reshape(_NC * _NT, _NG32, _GB, _K)
    ones = jnp.ones((_K, _W), jnp.float32)
    zeros = jnp.zeros((_P, _W), jnp.float32)
    xp = jnp.pad(x, ((0, _P - _N), (0, 0)))

    degp = _make_degree()(dst32, ones, zeros)
    hs1, dinv = _tc_pre(xp, degp)

    a1 = _make_agg_split()(hs1, zeros, src32, dst32)
    hs2 = _tc_conv([a1], dinv, Wc1, bc1, 2, "sum")
    a2 = _make_agg(2)(hs2[0], hs2[1], src16, dst16)
    hs3 = _tc_conv(a2, dinv, Wc2, bc2, 4, "cat")
    a3 = _make_agg(4)(hs3[0], hs3[1], hs3[2], hs3[3], src16, dst16)
    out = _tc_final(a3, dinv, Wc3, bc3, Wl1, bl1, Wl2, bl2, Wl3, bl3)
    return out[:_N]


# speedup vs baseline: 1.1918x; 1.0223x over previous
import functools

import jax
import jax.numpy as jnp
from jax import lax
from jax.experimental import pallas as pl
from jax.experimental.pallas import tpu as pltpu
from jax.experimental.pallas import tpu_sc as plsc

_N = 10000
_E = 320000
_P = 10240
_NT = 16
_NC = 2
_K = 100
_W = 128

_NB = _E // _NT // _K
_NB2 = _E // (_NC * _NT) // _K
_GB = 50
_NG16 = _NB // _GB
_NG32 = _NB2 // _GB
_RPT = _P // _NT


@functools.cache
def _get_mesh():
    return plsc.VectorSubcoreMesh(core_axis_name="c", subcore_axis_name="s",
                                  num_cores=_NC, num_subcores=_NT)


def _sweep(hs_c, acc, srcb, dstb, rows0, rows1, sem0, sem1, nb):
    pltpu.async_copy(hs_c.at[srcb.at[0]], rows0, sem0)
    half = nb // 2

    def pair(t, carry):
        i0 = 2 * t
        pltpu.async_copy(hs_c.at[srcb.at[i0 + 1]], rows1, sem1)
        pltpu.make_async_copy(hs_c.at[srcb.at[i0]], rows0, sem0).wait()
        pltpu.sync_copy(rows0, acc.at[dstb.at[i0]], add=True)

        @pl.when(t + 1 < half)
        def _():
            pltpu.async_copy(hs_c.at[srcb.at[i0 + 2]], rows0, sem0)

        pltpu.make_async_copy(hs_c.at[srcb.at[i0 + 1]], rows1, sem1).wait()
        pltpu.sync_copy(rows1, acc.at[dstb.at[i0 + 1]], add=True)
        return carry

    lax.fori_loop(0, half, pair, 0)


@functools.cache
def _make_degree():
    def body(dst_hbm, ones_hbm, zeros_hbm, out_hbm, dstb, onesb, acc):
        cid = lax.axis_index("c")
        sid = lax.axis_index("s")
        wid = cid * _NT + sid
        pltpu.sync_copy(ones_hbm, onesb)
        r0 = sid * _RPT
        pltpu.sync_copy(zeros_hbm.at[pl.ds(r0, _RPT), :], acc.at[pl.ds(r0, _RPT), :])
        plsc.subcore_barrier()

        def step(i, carry):
            pltpu.sync_copy(onesb, acc.at[dstb.at[i]], add=True)
            return carry

        for g in range(_NG32):
            pltpu.sync_copy(dst_hbm.at[wid, g], dstb)
            lax.fori_loop(0, _GB, step, 0)
        plsc.subcore_barrier()
        pltpu.sync_copy(acc.at[pl.ds(r0, _RPT), :], out_hbm.at[cid, pl.ds(r0, _RPT), :])

    return pl.kernel(
        body,
        out_type=jax.ShapeDtypeStruct((_NC, _P, _W), jnp.float32),
        mesh=_get_mesh(),
        scratch_types=[
            pltpu.VMEM((_GB, _K), jnp.int32),
            pltpu.VMEM((_K, _W), jnp.float32),
            pltpu.VMEM_SHARED((_P, _W), jnp.float32),
        ],
    )


@functools.cache
def _make_agg_split():
    def body(hs_hbm, zeros_hbm, src_hbm, dst_hbm, out_hbm,
             srcb, dstb, rows0, rows1, acc, sem0, sem1):
        cid = lax.axis_index("c")
        sid = lax.axis_index("s")
        wid = cid * _NT + sid
        r0 = sid * _RPT

        @pl.when(cid == 0)
        def _():
            pltpu.sync_copy(hs_hbm.at[pl.ds(r0, _RPT), :], acc.at[pl.ds(r0, _RPT), :])

        @pl.when(cid == 1)
        def _():
            pltpu.sync_copy(zeros_hbm.at[pl.ds(r0, _RPT), :], acc.at[pl.ds(r0, _RPT), :])

        plsc.subcore_barrier()
        for g in range(_NG32):
            pltpu.sync_copy(src_hbm.at[wid, g], srcb)
            pltpu.sync_copy(dst_hbm.at[wid, g], dstb)
            _sweep(hs_hbm, acc, srcb, dstb, rows0, rows1, sem0, sem1, _GB)
        plsc.subcore_barrier()
        pltpu.sync_copy(acc.at[pl.ds(r0, _RPT), :], out_hbm.at[cid, pl.ds(r0, _RPT), :])

    return pl.kernel(
        body,
        out_type=jax.ShapeDtypeStruct((_NC, _P, _W), jnp.float32),
        mesh=_get_mesh(),
        scratch_types=[
            pltpu.VMEM((_GB, _K), jnp.int32),
            pltpu.VMEM((_GB, _K), jnp.int32),
            pltpu.VMEM((_K, _W), jnp.float32),
            pltpu.VMEM((_K, _W), jnp.float32),
            pltpu.VMEM_SHARED((_P, _W), jnp.float32),
            pltpu.SemaphoreType.DMA,
            pltpu.SemaphoreType.DMA,
        ],
    )


@functools.cache
def _make_agg(C):
    Cpc = C // _NC

    def body(*refs):
        hs = refs[0:C]
        src_hbm = refs[C]
        dst_hbm = refs[C + 1]
        outs = refs[C + 2: 2 * C + 2]
        srcb, dstb, rows0, rows1, acc, sem0, sem1 = refs[2 * C + 2:]
        cid = lax.axis_index("c")
        sid = lax.axis_index("s")
        r0 = sid * _RPT

        def do_chunk(hs_c, out_c):
            pltpu.sync_copy(hs_c.at[pl.ds(r0, _RPT), :], acc.at[pl.ds(r0, _RPT), :])
            plsc.subcore_barrier()
            for g in range(_NG16):
                pltpu.sync_copy(src_hbm.at[sid, g], srcb)
                pltpu.sync_copy(dst_hbm.at[sid, g], dstb)
                _sweep(hs_c, acc, srcb, dstb, rows0, rows1, sem0, sem1, _GB)
            plsc.subcore_barrier()
            pltpu.sync_copy(acc.at[pl.ds(r0, _RPT), :], out_c.at[pl.ds(r0, _RPT), :])

        for core in range(_NC):
            @pl.when(cid == core)
            def _(core=core):
                for j in range(Cpc):
                    do_chunk(hs[core * Cpc + j], outs[core * Cpc + j])

    return pl.kernel(
        body,
        out_type=tuple(jax.ShapeDtypeStruct((_P, _W), jnp.float32) for _ in range(C)),
        mesh=_get_mesh(),
        scratch_types=[
            pltpu.VMEM((_GB, _K), jnp.int32),
            pltpu.VMEM((_GB, _K), jnp.int32),
            pltpu.VMEM((_K, _W), jnp.float32),
            pltpu.VMEM((_K, _W), jnp.float32),
            pltpu.VMEM_SHARED((_P, _W), jnp.float32),
            pltpu.SemaphoreType.DMA,
            pltpu.SemaphoreType.DMA,
        ],
    )


_BN = 1024
_G = _P // _BN


def _tc_pre(x, degp):
    def body(x_ref, d_ref, o_hs, o_dinv):
        deg = d_ref[0][:, :1] + d_ref[1][:, :1] + 1.0
        dinv = lax.rsqrt(deg)
        o_dinv[...] = dinv
        o_hs[...] = x_ref[...] * dinv

    return pl.pallas_call(
        body,
        grid=(_G,),
        in_specs=[
            pl.BlockSpec((_BN, 128), lambda i: (i, 0)),
            pl.BlockSpec((_NC, _BN, _W), lambda i: (0, i, 0)),
        ],
        out_specs=[
            pl.BlockSpec((_BN, 128), lambda i: (i, 0)),
            pl.BlockSpec((_BN, 1), lambda i: (i, 0)),
        ],
        out_shape=[
            jax.ShapeDtypeStruct((_P, 128), jnp.float32),
            jax.ShapeDtypeStruct((_P, 1), jnp.float32),
        ],
    )(x, degp)


def _ff(h, w_ref, b_ref):
    h = lax.dot_general(h, w_ref[...], (((1,), (1,)), ((), ())),
                        preferred_element_type=jnp.float32)
    return jnp.maximum(h + b_ref[...], 0.0)


def _tc_conv(a_parts, dinv, Wt, bt, C_out, combine):
    C_in = len(a_parts)
    d_out, d_in = Wt.shape
    W_in = d_in if combine == "sum" else d_in // C_in

    def body(*refs):
        a_refs = refs[:C_in]
        d_ref, w_ref, b_ref = refs[C_in:C_in + 3]
        outs = refs[C_in + 3:]
        if combine == "sum":
            a = a_refs[0][0] + a_refs[0][1]
        else:
            a = jnp.concatenate([r[...] for r in a_refs], axis=1)
        dinv = d_ref[...]
        h = _ff(a * dinv, w_ref, b_ref) * dinv
        for c, o in enumerate(outs):
            o[...] = h[:, c * _W:(c + 1) * _W]

    if combine == "sum":
        a_specs = [pl.BlockSpec((_NC, _BN, W_in), lambda i: (0, i, 0))]
    else:
        a_specs = [pl.BlockSpec((_BN, W_in), lambda i: (i, 0))] * C_in
    in_specs = a_specs + [
        pl.BlockSpec((_BN, 1), lambda i: (i, 0)),
        pl.BlockSpec((d_out, d_in), lambda i: (0, 0)),
        pl.BlockSpec((1, d_out), lambda i: (0, 0)),
    ]
    return pl.pallas_call(
        body,
        grid=(_G,),
        in_specs=in_specs,
        out_specs=[pl.BlockSpec((_BN, _W), lambda i: (i, 0))] * C_out,
        out_shape=[jax.ShapeDtypeStruct((_P, _W), jnp.float32)] * C_out,
    )(*a_parts, dinv, Wt, bt.reshape(1, -1))


def _tc_final(a_chunks, dinv, Wc3, bc3, Wl1, bl1, Wl2, bl2, Wl3, bl3):
    def body(a0, a1, a2, a3, d_ref, w3, b3, w1, b1, w2, b2, wl, bl, o):
        a = jnp.concatenate([a0[...], a1[...], a2[...], a3[...]], axis=1)
        h = _ff(a * d_ref[...], w3, b3)
        h = _ff(h, w1, b1)
        h = _ff(h, w2, b2)
        o[...] = _ff(h, wl, bl)

    def wspec(shape):
        return pl.BlockSpec(shape, lambda i: (0, 0))

    in_specs = (
        [pl.BlockSpec((_BN, _W), lambda i: (i, 0))] * 4 + [
            pl.BlockSpec((_BN, 1), lambda i: (i, 0)),
            wspec((1024, 512)), wspec((1, 1024)),
            wspec((512, 1024)), wspec((1, 512)),
            wspec((256, 512)), wspec((1, 256)),
            wspec((128, 256)), wspec((1, 128)),
        ]
    )
    return pl.pallas_call(
        body,
        grid=(_G,),
        in_specs=in_specs,
        out_specs=pl.BlockSpec((_BN, 128), lambda i: (i, 0)),
        out_shape=jax.ShapeDtypeStruct((_P, 128), jnp.float32),
    )(*a_chunks, dinv, Wc3, bc3.reshape(1, -1), Wl1, bl1.reshape(1, -1),
      Wl2, bl2.reshape(1, -1), Wl3, bl3.reshape(1, -1))


def kernel(x, edge_index, Wc1, bc1, Wc2, bc2, Wc3, bc3,
           Wl1, bl1, Wl2, bl2, Wl3, bl3):
    src16 = edge_index[0].reshape(_NT, _NG16, _GB, _K)
    dst16 = edge_index[1].reshape(_NT, _NG16, _GB, _K)
    src32 = edge_index[0].reshape(_NC * _NT, _NG32, _GB, _K)
    dst32 = edge_index[1].reshape(_NC * _NT, _NG32, _GB, _K)
    ones = jnp.ones((_K, _W), jnp.float32)
    zeros = jnp.zeros((_P, _W), jnp.float32)
    xp = jnp.pad(x, ((0, _P - _N), (0, 0)))

    degp = _make_degree()(dst32, ones, zeros)
    hs1, dinv = _tc_pre(xp, degp)

    a1 = _make_agg_split()(hs1, zeros, src32, dst32)
    hs2 = _tc_conv([a1], dinv, Wc1, bc1, 2, "sum")
    a2 = _make_agg(2)(hs2[0], hs2[1], src16, dst16)
    hs3 = _tc_conv(a2, dinv, Wc2, bc2, 4, "cat")
    a3 = _make_agg(4)(hs3[0], hs3[1], hs3[2], hs3[3], src16, dst16)
    out = _tc_final(a3, dinv, Wc3, bc3, Wl1, bl1, Wl2, bl2, Wl3, bl3)
    return out[:_N]
